# Initial kernel scaffold; baseline (speedup 1.0000x reference)
import jax
import jax.numpy as jnp
from jax.experimental import pallas as pl


def kernel(x, emb_weight):
    raise NotImplementedError("write your pallas kernel here")


# speedup vs baseline: 1.6675x; 1.6675x over previous
import jax
import jax.numpy as jnp
from jax.experimental import pallas as pl
from jax.experimental.pallas import tpu as pltpu

_BR = 512


def _add_body(x_ref, emb_ref, out_ref):
    out_ref[0] = x_ref[0] + emb_ref[...]


def kernel(x, emb_weight):
    batch, num_patches, dim = x.shape
    nb = num_patches // _BR
    return pl.pallas_call(
        _add_body,
        grid=(nb, batch),
        in_specs=[
            pl.BlockSpec((1, _BR, dim), lambda i, b: (b, i, 0)),
            pl.BlockSpec((_BR, dim), lambda i, b: (i, 0)),
        ],
        out_specs=pl.BlockSpec((1, _BR, dim), lambda i, b: (b, i, 0)),
        out_shape=jax.ShapeDtypeStruct(x.shape, x.dtype),
        compiler_params=pltpu.CompilerParams(
            dimension_semantics=("arbitrary", "arbitrary"),
        ),
    )(x, emb_weight)


# speedup vs baseline: 1.8554x; 1.1126x over previous
import jax
import jax.numpy as jnp
from jax.experimental import pallas as pl
from jax.experimental.pallas import tpu as pltpu

_BR = 1024


def _add_body(x_ref, emb_ref, out_ref):
    out_ref[0] = x_ref[0] + emb_ref[...]


def kernel(x, emb_weight):
    batch, num_patches, dim = x.shape
    nb = num_patches // _BR
    return pl.pallas_call(
        _add_body,
        grid=(nb, batch),
        in_specs=[
            pl.BlockSpec((1, _BR, dim), lambda i, b: (b, i, 0)),
            pl.BlockSpec((_BR, dim), lambda i, b: (i, 0)),
        ],
        out_specs=pl.BlockSpec((1, _BR, dim), lambda i, b: (b, i, 0)),
        out_shape=jax.ShapeDtypeStruct(x.shape, x.dtype),
        compiler_params=pltpu.CompilerParams(
            dimension_semantics=("arbitrary", "arbitrary"),
        ),
    )(x, emb_weight)


# speedup vs baseline: 1.9673x; 1.0603x over previous
import jax
import jax.numpy as jnp
from jax.experimental import pallas as pl
from jax.experimental.pallas import tpu as pltpu

_BR = 2048


def _add_body(x_ref, emb_ref, out_ref):
    out_ref[0] = x_ref[0] + emb_ref[...]


def kernel(x, emb_weight):
    batch, num_patches, dim = x.shape
    nb = num_patches // _BR
    return pl.pallas_call(
        _add_body,
        grid=(nb, batch),
        in_specs=[
            pl.BlockSpec((1, _BR, dim), lambda i, b: (b, i, 0)),
            pl.BlockSpec((_BR, dim), lambda i, b: (i, 0)),
        ],
        out_specs=pl.BlockSpec((1, _BR, dim), lambda i, b: (b, i, 0)),
        out_shape=jax.ShapeDtypeStruct(x.shape, x.dtype),
        compiler_params=pltpu.CompilerParams(
            dimension_semantics=("arbitrary", "arbitrary"),
        ),
    )(x, emb_weight)
